# baseline (speedup 1.0000x reference)
import functools

import jax
import jax.numpy as jnp
from jax import lax
from jax.experimental import pallas as pl
from jax.experimental.pallas import tpu as pltpu
from jax.experimental.pallas import tpu_sc as plsc

_LANES = 16
_NBUF = 8


@functools.lru_cache(maxsize=None)
def _build(B, S, H, NC, NS):
    NW = NC * NS
    s_per_w = S // NW
    C = max(8, s_per_w // 4)
    n_sub = s_per_w // C
    NCH = B * n_sub

    mesh = plsc.VectorSubcoreMesh(core_axis_name="c", subcore_axis_name="s")

    @functools.partial(
        pl.kernel,
        out_type=jax.ShapeDtypeStruct((B * S, H), jnp.float32),
        mesh=mesh,
        scratch_types=[
            *[pltpu.VMEM((C,), jnp.int32) for _ in range(NCH)],
            *[pltpu.VMEM((C, H), jnp.float32) for _ in range(_NBUF)],
            pltpu.VMEM((s_per_w, H), jnp.float32),
            *[pltpu.SemaphoreType.DMA for _ in range(2 * _NBUF + 2)],
        ],
    )
    def emb(ids_hbm, tok_hbm, pos_hbm, out_hbm, *refs):
        idx = refs[:NCH]
        tok = refs[NCH:NCH + _NBUF]
        pos_v = refs[NCH + _NBUF]
        gsem = refs[NCH + _NBUF + 1:NCH + _NBUF + 1 + _NBUF]
        ssem = refs[NCH + _NBUF + 1 + _NBUF:NCH + 1 + 3 * _NBUF]
        isem, psem = refs[NCH + 1 + 3 * _NBUF:]

        wid = lax.axis_index("s") * NC + lax.axis_index("c")
        s_base = wid * s_per_w

        def flat_base(c):
            b, sub = divmod(c, n_sub)
            return b * S + s_base + sub * C

        def idx_src(c):
            return ids_hbm.at[c // n_sub, pl.ds(s_base + (c % n_sub) * C, C)]

        idx0_cp = pltpu.async_copy(idx_src(0), idx[0], isem)
        gathers = [None] * NCH
        stores = [None] * NCH
        idx0_cp.wait()
        gathers[0] = pltpu.async_copy(tok_hbm.at[idx[0]], tok[0], gsem[0])
        pos_cp = pltpu.async_copy(pos_hbm.at[pl.ds(s_base, s_per_w)], pos_v,
                                  psem)
        idx_cps = [pltpu.async_copy(idx_src(c), idx[c], isem)
                   for c in range(1, NCH)]
        for cp in idx_cps:
            cp.wait()
        for c in range(1, min(_NBUF - 1, NCH)):
            gathers[c] = pltpu.async_copy(tok_hbm.at[idx[c]], tok[c % _NBUF],
                                          gsem[c % _NBUF])

        def row_add(t_ref, p_off, r, _):
            for j in range(H // _LANES):
                sl = pl.ds(j * _LANES, _LANES)
                plsc.addupdate(t_ref.at[r, sl], pos_v[p_off + r, sl])
            return 0

        for c in range(NCH):
            buf = c % _NBUF
            nc = c + _NBUF - 1
            if nc < NCH:
                nbuf = nc % _NBUF
                if c >= 1:
                    stores[c - 1].wait()
                gathers[nc] = pltpu.async_copy(tok_hbm.at[idx[nc]], tok[nbuf],
                                               gsem[nbuf])
            if c == 0:
                pos_cp.wait()
            gathers[c].wait()
            p_off = (c % n_sub) * C
            lax.fori_loop(0, C, functools.partial(row_add, tok[buf], p_off), 0)
            stores[c] = pltpu.async_copy(
                tok[buf], out_hbm.at[pl.ds(flat_base(c), C)], ssem[buf])
        for c in range(max(0, NCH - _NBUF), NCH):
            stores[c].wait()

    return emb


def kernel(input_ids, token_table, pos_table):
    B, S = input_ids.shape
    H = token_table.shape[1]
    info = plsc.get_sparse_core_info()
    emb = _build(B, S, H, info.num_cores, info.num_subcores)
    ids = input_ids.astype(jnp.int32)
    out = emb(ids, token_table, pos_table)
    return out.reshape(B, S, H)


# speedup vs baseline: 1.0312x; 1.0312x over previous
import functools

import jax
import jax.numpy as jnp
from jax import lax
from jax.experimental import pallas as pl
from jax.experimental.pallas import tpu as pltpu
from jax.experimental.pallas import tpu_sc as plsc

_LANES = 16
_NBUF = 5


@functools.lru_cache(maxsize=None)
def _build(B, S, H, NC, NS):
    NW = NC * NS
    s_per_w = S // NW
    C = max(8, s_per_w // 2)
    n_sub = s_per_w // C
    NCH = B * n_sub

    mesh = plsc.VectorSubcoreMesh(core_axis_name="c", subcore_axis_name="s")

    @functools.partial(
        pl.kernel,
        out_type=jax.ShapeDtypeStruct((B * S, H), jnp.float32),
        mesh=mesh,
        scratch_types=[
            *[pltpu.VMEM((C,), jnp.int32) for _ in range(NCH)],
            *[pltpu.VMEM((C, H), jnp.float32) for _ in range(_NBUF)],
            pltpu.VMEM((s_per_w, H), jnp.float32),
            *[pltpu.SemaphoreType.DMA for _ in range(2 * _NBUF + 2)],
        ],
    )
    def emb(ids_hbm, tok_hbm, pos_hbm, out_hbm, *refs):
        idx = refs[:NCH]
        tok = refs[NCH:NCH + _NBUF]
        pos_v = refs[NCH + _NBUF]
        gsem = refs[NCH + _NBUF + 1:NCH + _NBUF + 1 + _NBUF]
        ssem = refs[NCH + _NBUF + 1 + _NBUF:NCH + 1 + 3 * _NBUF]
        isem, psem = refs[NCH + 1 + 3 * _NBUF:]

        wid = lax.axis_index("s") * NC + lax.axis_index("c")
        s_base = wid * s_per_w

        def flat_base(c):
            b, sub = divmod(c, n_sub)
            return b * S + s_base + sub * C

        def idx_src(c):
            return ids_hbm.at[c // n_sub, pl.ds(s_base + (c % n_sub) * C, C)]

        idx0_cp = pltpu.async_copy(idx_src(0), idx[0], isem)
        gathers = [None] * NCH
        stores = [None] * NCH
        idx0_cp.wait()
        gathers[0] = pltpu.async_copy(tok_hbm.at[idx[0]], tok[0], gsem[0])
        pos_cps = [None] * n_sub
        pos_cps[0] = pltpu.async_copy(pos_hbm.at[pl.ds(s_base, C)],
                                      pos_v.at[pl.ds(0, C)], psem)
        idx_cps = [pltpu.async_copy(idx_src(c), idx[c], isem)
                   for c in range(1, NCH)]
        for cp in idx_cps:
            cp.wait()
        for c in range(1, min(_NBUF - 1, NCH)):
            gathers[c] = pltpu.async_copy(tok_hbm.at[idx[c]], tok[c % _NBUF],
                                          gsem[c % _NBUF])
        for k in range(1, n_sub):
            pos_cps[k] = pltpu.async_copy(
                pos_hbm.at[pl.ds(s_base + k * C, C)],
                pos_v.at[pl.ds(k * C, C)], isem)

        def row_add(t_ref, p_off, r, _):
            for j in range(H // _LANES):
                sl = pl.ds(j * _LANES, _LANES)
                plsc.addupdate(t_ref.at[r, sl], pos_v[p_off + r, sl])
            return 0

        for c in range(NCH):
            buf = c % _NBUF
            nc = c + _NBUF - 1
            if nc < NCH:
                nbuf = nc % _NBUF
                if c >= 1:
                    stores[c - 1].wait()
                gathers[nc] = pltpu.async_copy(tok_hbm.at[idx[nc]], tok[nbuf],
                                               gsem[nbuf])
            if c < n_sub:
                pos_cps[c].wait()
            gathers[c].wait()
            p_off = (c % n_sub) * C
            lax.fori_loop(0, C, functools.partial(row_add, tok[buf], p_off), 0)
            stores[c] = pltpu.async_copy(
                tok[buf], out_hbm.at[pl.ds(flat_base(c), C)], ssem[buf])
        for c in range(max(0, NCH - _NBUF), NCH):
            stores[c].wait()

    return emb


def kernel(input_ids, token_table, pos_table):
    B, S = input_ids.shape
    H = token_table.shape[1]
    info = plsc.get_sparse_core_info()
    emb = _build(B, S, H, info.num_cores, info.num_subcores)
    ids = input_ids.astype(jnp.int32)
    out = emb(ids, token_table, pos_table)
    return out.reshape(B, S, H)


# speedup vs baseline: 1.0767x; 1.0442x over previous
import functools

import jax
import jax.numpy as jnp
from jax import lax
from jax.experimental import pallas as pl
from jax.experimental.pallas import tpu as pltpu
from jax.experimental.pallas import tpu_sc as plsc

_LANES = 16
_NBUF = 5


@functools.lru_cache(maxsize=None)
def _build(B, S, H, NC, NS):
    NW = NC * NS
    s_per_w = S // NW
    C = max(8, s_per_w // 2)
    n_sub = s_per_w // C
    NCH = B * n_sub

    mesh = plsc.VectorSubcoreMesh(core_axis_name="c", subcore_axis_name="s")

    @functools.partial(
        pl.kernel,
        out_type=jax.ShapeDtypeStruct((B * S, H), jnp.float32),
        mesh=mesh,
        scratch_types=[
            *[pltpu.VMEM((C,), jnp.int32) for _ in range(NCH)],
            *[pltpu.VMEM((C, H), jnp.float32) for _ in range(_NBUF)],
            pltpu.VMEM((s_per_w, H), jnp.float32),
            *[pltpu.SemaphoreType.DMA for _ in range(2 * _NBUF + 2)],
        ],
    )
    def emb(ids_hbm, tok_hbm, pos_hbm, out_hbm, *refs):
        idx = refs[:NCH]
        tok = refs[NCH:NCH + _NBUF]
        pos_v = refs[NCH + _NBUF]
        gsem = refs[NCH + _NBUF + 1:NCH + _NBUF + 1 + _NBUF]
        ssem = refs[NCH + _NBUF + 1 + _NBUF:NCH + 1 + 3 * _NBUF]
        isem, psem = refs[NCH + 1 + 3 * _NBUF:]

        wid = lax.axis_index("s") * NC + lax.axis_index("c")
        s_base = wid * s_per_w

        def flat_base(c):
            b, sub = divmod(c, n_sub)
            return b * S + s_base + sub * C

        def idx_src(c):
            return ids_hbm.at[c // n_sub, pl.ds(s_base + (c % n_sub) * C, C)]

        idx0_cp = pltpu.async_copy(idx_src(0), idx[0], isem)
        gathers = [None] * NCH
        stores = [None] * NCH
        idx0_cp.wait()
        gathers[0] = pltpu.async_copy(tok_hbm.at[idx[0]], tok[0], gsem[0])
        pos_cps = [None] * n_sub
        pos_cps[0] = pltpu.async_copy(pos_hbm.at[pl.ds(s_base, C)],
                                      pos_v.at[pl.ds(0, C)], psem)
        idx_cps = [pltpu.async_copy(idx_src(c), idx[c], isem)
                   for c in range(1, NCH)]
        for cp in idx_cps:
            cp.wait()
        lookahead = _NBUF - 2
        for c in range(1, min(lookahead, NCH)):
            gathers[c] = pltpu.async_copy(tok_hbm.at[idx[c]], tok[c % _NBUF],
                                          gsem[c % _NBUF])
        for k in range(1, n_sub):
            pos_cps[k] = pltpu.async_copy(
                pos_hbm.at[pl.ds(s_base + k * C, C)],
                pos_v.at[pl.ds(k * C, C)], isem)

        def row_add(t_ref, p_off, r, _):
            for j in range(H // _LANES):
                sl = pl.ds(j * _LANES, _LANES)
                plsc.addupdate(t_ref.at[r, sl], pos_v[p_off + r, sl])
            return 0

        drained = set()
        for c in range(NCH):
            buf = c % _NBUF
            nc = c + lookahead
            if nc < NCH:
                pv = nc - _NBUF
                if pv >= 0:
                    stores[pv].wait()
                    drained.add(pv)
                gathers[nc] = pltpu.async_copy(
                    tok_hbm.at[idx[nc]], tok[nc % _NBUF], gsem[nc % _NBUF])
            if c < n_sub:
                pos_cps[c].wait()
            gathers[c].wait()
            p_off = (c % n_sub) * C
            lax.fori_loop(0, C, functools.partial(row_add, tok[buf], p_off), 0)
            stores[c] = pltpu.async_copy(
                tok[buf], out_hbm.at[pl.ds(flat_base(c), C)], ssem[buf])
        for c in range(NCH):
            if c not in drained:
                stores[c].wait()

    return emb


def kernel(input_ids, token_table, pos_table):
    B, S = input_ids.shape
    H = token_table.shape[1]
    info = plsc.get_sparse_core_info()
    emb = _build(B, S, H, info.num_cores, info.num_subcores)
    ids = input_ids.astype(jnp.int32)
    out = emb(ids, token_table, pos_table)
    return out.reshape(B, S, H)


# speedup vs baseline: 1.0997x; 1.0213x over previous
import functools

import jax
import jax.numpy as jnp
from jax import lax
from jax.experimental import pallas as pl
from jax.experimental.pallas import tpu as pltpu
from jax.experimental.pallas import tpu_sc as plsc

_LANES = 16
_NBUF = 5


@functools.lru_cache(maxsize=None)
def _build(B, S, H, NC, NS):
    NW = NC * NS
    s_per_w = S // NW
    C = max(8, s_per_w // 2)
    n_sub = s_per_w // C
    NCH = B * n_sub

    mesh = plsc.VectorSubcoreMesh(core_axis_name="c", subcore_axis_name="s")

    @functools.partial(
        pl.kernel,
        out_type=jax.ShapeDtypeStruct((B * S, H), jnp.float32),
        mesh=mesh,
        scratch_types=[
            *[pltpu.VMEM((C,), jnp.int32) for _ in range(NCH)],
            *[pltpu.VMEM((C, H), jnp.float32) for _ in range(_NBUF)],
            pltpu.VMEM((s_per_w, H), jnp.float32),
            *[pltpu.SemaphoreType.DMA for _ in range(2 * _NBUF + 2)],
        ],
    )
    def emb(ids_hbm, tok_hbm, pos_hbm, out_hbm, *refs):
        idx = refs[:NCH]
        tok = refs[NCH:NCH + _NBUF]
        pos_v = refs[NCH + _NBUF]
        gsem = refs[NCH + _NBUF + 1:NCH + _NBUF + 1 + _NBUF]
        ssem = refs[NCH + _NBUF + 1 + _NBUF:NCH + 1 + 3 * _NBUF]
        isem, psem = refs[NCH + 1 + 3 * _NBUF:]

        wid = lax.axis_index("s") * NC + lax.axis_index("c")
        s_base = wid * s_per_w

        def flat_base(c):
            b, sub = divmod(c, n_sub)
            return b * S + s_base + sub * C

        def idx_src(c):
            return ids_hbm.at[c // n_sub, pl.ds(s_base + (c % n_sub) * C, C)]

        idx0_cp = pltpu.async_copy(idx_src(0), idx[0], isem)
        gathers = [None] * NCH
        stores = [None] * NCH
        idx0_cp.wait()
        gathers[0] = pltpu.async_copy(tok_hbm.at[idx[0]], tok[0], gsem[0])
        pos_cps = [None] * n_sub
        pos_cps[0] = pltpu.async_copy(pos_hbm.at[pl.ds(s_base, C)],
                                      pos_v.at[pl.ds(0, C)], psem)
        idx_cps = [pltpu.async_copy(idx_src(c), idx[c], isem)
                   for c in range(1, NCH)]
        for cp in idx_cps:
            cp.wait()
        lookahead = _NBUF - 3
        for c in range(1, min(lookahead, NCH)):
            gathers[c] = pltpu.async_copy(tok_hbm.at[idx[c]], tok[c % _NBUF],
                                          gsem[c % _NBUF])
        for k in range(1, n_sub):
            pos_cps[k] = pltpu.async_copy(
                pos_hbm.at[pl.ds(s_base + k * C, C)],
                pos_v.at[pl.ds(k * C, C)], isem)

        def row_add(t_ref, p_off, r, _):
            for j in range(H // _LANES):
                sl = pl.ds(j * _LANES, _LANES)
                plsc.addupdate(t_ref.at[r, sl], pos_v[p_off + r, sl])
            return 0

        drained = set()
        for c in range(NCH):
            buf = c % _NBUF
            nc = c + lookahead
            if nc < NCH:
                pv = nc - _NBUF
                if pv >= 0:
                    stores[pv].wait()
                    drained.add(pv)
                gathers[nc] = pltpu.async_copy(
                    tok_hbm.at[idx[nc]], tok[nc % _NBUF], gsem[nc % _NBUF])
            if c < n_sub:
                pos_cps[c].wait()
            gathers[c].wait()
            p_off = (c % n_sub) * C
            lax.fori_loop(0, C, functools.partial(row_add, tok[buf], p_off), 0)
            stores[c] = pltpu.async_copy(
                tok[buf], out_hbm.at[pl.ds(flat_base(c), C)], ssem[buf])
        for c in range(NCH):
            if c not in drained:
                stores[c].wait()

    return emb


def kernel(input_ids, token_table, pos_table):
    B, S = input_ids.shape
    H = token_table.shape[1]
    info = plsc.get_sparse_core_info()
    emb = _build(B, S, H, info.num_cores, info.num_subcores)
    ids = input_ids.astype(jnp.int32)
    out = emb(ids, token_table, pos_table)
    return out.reshape(B, S, H)
